# baseline (speedup 1.0000x reference)
import dataclasses
import functools

import jax
import jax.numpy as jnp
from jax import lax
from jax.experimental import pallas as pl
from jax.experimental.pallas import tpu as pltpu
from jax.experimental.pallas import tpu_sc as plsc

BATCH = 16384
D = 32
NC = 2
NS = 16
L = 16
NW = NC * NS
BPW = BATCH // NW
CHUNK = 128
NCHUNK = BPW // CHUNK
NGROUP = BPW // L


def _sc_dot_kernel(uidx_hbm, iidx_hbm, utab_hbm, itab_hbm, out_hbm,
                   uidx_v, iidx_v, urows_v, irows_v, out_v, usem, isem):
    wid = lax.axis_index("s") * NC + lax.axis_index("c")
    base = wid * BPW

    pltpu.sync_copy(uidx_hbm.at[wid], uidx_v)
    pltpu.sync_copy(iidx_hbm.at[wid], iidx_v)

    copies = []
    for k in range(NCHUNK):
        dst = pl.ds(k * CHUNK, CHUNK)
        copies.append(pltpu.async_copy(utab_hbm.at[uidx_v.at[k]],
                                       urows_v.at[dst], usem))
        copies.append(pltpu.async_copy(itab_hbm.at[iidx_v.at[k]],
                                       irows_v.at[dst], isem))
    for c in copies:
        c.wait()

    iota16 = lax.iota(jnp.int32, L)

    @pl.loop(0, NGROUP)
    def _(g):
        rows = g * L + iota16
        acc = jnp.zeros((L,), jnp.float32)
        for d in range(D):
            col = jnp.full((L,), d, jnp.int32)
            u = plsc.load_gather(urows_v, [rows, col])
            v = plsc.load_gather(irows_v, [rows, col])
            acc = acc + u * v
        out_v[pl.ds(g * L, L)] = acc

    pltpu.sync_copy(out_v, out_hbm.at[pl.ds(base, BPW)])


def kernel(user_indices, item_indices, user_table, item_table):
    mesh = plsc.VectorSubcoreMesh(core_axis_name="c", subcore_axis_name="s")
    cp = pltpu.CompilerParams()
    for fld, val in (("needs_layout_passes", False),
                     ("use_tc_tiling_on_sc", False)):
        if fld in pltpu.CompilerParams.__dataclass_fields__:
            cp = dataclasses.replace(cp, **{fld: val})
    sc_call = functools.partial(
        pl.kernel,
        out_type=jax.ShapeDtypeStruct((BATCH,), jnp.float32),
        mesh=mesh,
        compiler_params=cp,
        scratch_types=[
            pltpu.VMEM((NCHUNK, CHUNK), jnp.int32),
            pltpu.VMEM((NCHUNK, CHUNK), jnp.int32),
            pltpu.VMEM((BPW, D), jnp.float32),
            pltpu.VMEM((BPW, D), jnp.float32),
            pltpu.VMEM((BPW,), jnp.float32),
            pltpu.SemaphoreType.DMA,
            pltpu.SemaphoreType.DMA,
        ],
    )(_sc_dot_kernel)
    uidx = user_indices.astype(jnp.int32).reshape(NW, NCHUNK, CHUNK)
    iidx = item_indices.astype(jnp.int32).reshape(NW, NCHUNK, CHUNK)
    return sc_call(uidx, iidx, user_table, item_table)


# speedup vs baseline: 1.7454x; 1.7454x over previous
import dataclasses
import functools

import jax
import jax.numpy as jnp
from jax import lax
from jax.experimental import pallas as pl
from jax.experimental.pallas import tpu as pltpu
from jax.experimental.pallas import tpu_sc as plsc

BATCH = 16384
D = 32
NC = 2
NS = 16
L = 16
NW = NC * NS
BPW = BATCH // NW
CHUNK = 128
NCHUNK = BPW // CHUNK
GPC = CHUNK // L


def _sc_dot_kernel(uR_hbm, um_hbm, iR_hbm, im_hbm, utab_hbm, itab_hbm,
                   out_hbm, uR_v, um_v, iR_v, im_v,
                   ubuf0, ubuf1, ibuf0, ibuf1, out_v, usem, isem):
    wid = lax.axis_index("s") * NC + lax.axis_index("c")
    base = wid * BPW

    pltpu.sync_copy(uR_hbm.at[pl.ds(base, BPW)], uR_v)
    pltpu.sync_copy(um_hbm.at[pl.ds(base, BPW)], um_v)
    pltpu.sync_copy(iR_hbm.at[pl.ds(base, BPW)], iR_v)
    pltpu.sync_copy(im_hbm.at[pl.ds(base, BPW)], im_v)

    ubufs = (ubuf0, ubuf1)
    ibufs = (ibuf0, ibuf1)

    def fire(c):
        sl = pl.ds(c * CHUNK, CHUNK)
        return (pltpu.async_copy(utab_hbm.at[uR_v.at[sl]], ubufs[c % 2], usem),
                pltpu.async_copy(itab_hbm.at[iR_v.at[sl]], ibufs[c % 2], isem))

    iota16 = lax.iota(jnp.int32, L)
    inflight = {0: fire(0), 1: fire(1)}

    for c in range(NCHUNK):
        for cp in inflight.pop(c):
            cp.wait()
        ub, ib = ubufs[c % 2], ibufs[c % 2]

        @pl.loop(0, GPC)
        def _(g):
            rows = g * L + iota16
            sl16 = lambda gg: pl.ds(c * CHUNK + gg * L, L)
            mu32 = um_v[sl16(g)] * 32
            mi32 = im_v[sl16(g)] * 32
            acc = jnp.zeros((L,), jnp.float32)
            for d in range(D):
                u = plsc.load_gather(ub, [rows, mu32 + d])
                v = plsc.load_gather(ib, [rows, mi32 + d])
                acc = acc + u * v
            out_v[sl16(g)] = acc

        if c + 2 < NCHUNK:
            inflight[c + 2] = fire(c + 2)

    pltpu.sync_copy(out_v, out_hbm.at[pl.ds(base, BPW)])


UNTILE_W = 8192
UNTILE_STEPS = -(-1000000 // UNTILE_W)
PACK_ROWS = UNTILE_W // 4
NPACK = UNTILE_STEPS * PACK_ROWS


def _untile_body(u_ref, i_ref, uo_ref, io_ref):
    for src, dst in ((u_ref, uo_ref), (i_ref, io_ref)):
        t = src[...].T
        dst[...] = jnp.concatenate(
            [t[j * PACK_ROWS:(j + 1) * PACK_ROWS] for j in range(4)], axis=1)


def _untile(uT, iT):
    spec_in = pl.BlockSpec((D, UNTILE_W), lambda i: (0, i))
    spec_out = pl.BlockSpec((PACK_ROWS, 4 * D), lambda i: (i, 0))
    return pl.pallas_call(
        _untile_body,
        grid=(UNTILE_STEPS,),
        in_specs=[spec_in, spec_in],
        out_specs=[spec_out, spec_out],
        out_shape=[jax.ShapeDtypeStruct((NPACK, 4 * D), jnp.float32)] * 2,
    )(uT, iT)


def kernel(user_indices, item_indices, user_table, item_table):
    mesh = plsc.VectorSubcoreMesh(core_axis_name="c", subcore_axis_name="s")
    cp = pltpu.CompilerParams()
    for fld, val in (("needs_layout_passes", False),
                     ("use_tc_tiling_on_sc", True)):
        if fld in pltpu.CompilerParams.__dataclass_fields__:
            cp = dataclasses.replace(cp, **{fld: val})
    sc_call = functools.partial(
        pl.kernel,
        out_type=jax.ShapeDtypeStruct((BATCH,), jnp.float32),
        mesh=mesh,
        compiler_params=cp,
        scratch_types=[
            pltpu.VMEM((BPW,), jnp.int32),
            pltpu.VMEM((BPW,), jnp.int32),
            pltpu.VMEM((BPW,), jnp.int32),
            pltpu.VMEM((BPW,), jnp.int32),
            pltpu.VMEM((CHUNK, 4 * D), jnp.float32),
            pltpu.VMEM((CHUNK, 4 * D), jnp.float32),
            pltpu.VMEM((CHUNK, 4 * D), jnp.float32),
            pltpu.VMEM((CHUNK, 4 * D), jnp.float32),
            pltpu.VMEM((BPW,), jnp.float32),
            pltpu.SemaphoreType.DMA,
            pltpu.SemaphoreType.DMA,
        ],
    )(_sc_dot_kernel)
    ui = user_indices.astype(jnp.int32)
    ii = item_indices.astype(jnp.int32)
    u128, i128 = _untile(user_table.T, item_table.T)
    uR = ((ui >> 13) << 11) + (ui & 2047)
    iR = ((ii >> 13) << 11) + (ii & 2047)
    um = (ui & 8191) >> 11
    im = (ii & 8191) >> 11
    return sc_call(uR, um, iR, im, u128, i128)


# speedup vs baseline: 1.7459x; 1.0002x over previous
import dataclasses
import functools

import jax
import jax.numpy as jnp
from jax import lax
from jax.experimental import pallas as pl
from jax.experimental.pallas import tpu as pltpu
from jax.experimental.pallas import tpu_sc as plsc

BATCH = 16384
D = 32
NC = 2
NS = 16
L = 16
NW = NC * NS
BPW = BATCH // NW
CHUNK = 128
NCHUNK = BPW // CHUNK
GPC = CHUNK // L


def _sc_dot_kernel(uR_hbm, um_hbm, iR_hbm, im_hbm, utab_hbm, itab_hbm,
                   out_hbm, uR_v, um_v, iR_v, im_v,
                   ubuf0, ubuf1, ibuf0, ibuf1, out_v, usem, isem):
    wid = lax.axis_index("s") * NC + lax.axis_index("c")
    base = wid * BPW

    pltpu.sync_copy(uR_hbm.at[pl.ds(base, BPW)], uR_v)
    pltpu.sync_copy(um_hbm.at[pl.ds(base, BPW)], um_v)
    pltpu.sync_copy(iR_hbm.at[pl.ds(base, BPW)], iR_v)
    pltpu.sync_copy(im_hbm.at[pl.ds(base, BPW)], im_v)

    ubufs = (ubuf0, ubuf1)
    ibufs = (ibuf0, ibuf1)

    def fire(c):
        sl = pl.ds(c * CHUNK, CHUNK)
        return (pltpu.async_copy(utab_hbm.at[uR_v.at[sl]], ubufs[c % 2], usem),
                pltpu.async_copy(itab_hbm.at[iR_v.at[sl]], ibufs[c % 2], isem))

    iota16 = lax.iota(jnp.int32, L)
    inflight = {0: fire(0), 1: fire(1)}

    for c in range(NCHUNK):
        for cp in inflight.pop(c):
            cp.wait()
        ub, ib = ubufs[c % 2], ibufs[c % 2]

        @pl.loop(0, GPC)
        def _(g):
            rows = g * L + iota16
            sl16 = lambda gg: pl.ds(c * CHUNK + gg * L, L)
            mu32 = um_v[sl16(g)] * 32
            mi32 = im_v[sl16(g)] * 32
            acc = jnp.zeros((L,), jnp.float32)
            for d in range(D):
                u = plsc.load_gather(ub, [rows, mu32 + d])
                v = plsc.load_gather(ib, [rows, mi32 + d])
                acc = acc + u * v
            out_v[sl16(g)] = acc

        if c + 2 < NCHUNK:
            inflight[c + 2] = fire(c + 2)

    pltpu.sync_copy(out_v, out_hbm.at[pl.ds(base, BPW)])


UNTILE_W = 8192
UNTILE_STEPS = -(-1000000 // UNTILE_W)
PACK_ROWS = UNTILE_W // 4
NPACK = UNTILE_STEPS * PACK_ROWS


def _untile_body(u_ref, i_ref, uo_ref, io_ref):
    eye = (lax.broadcasted_iota(jnp.int32, (D, D), 0) ==
           lax.broadcasted_iota(jnp.int32, (D, D), 1)).astype(jnp.float32)
    for src, dst in ((u_ref, uo_ref), (i_ref, io_ref)):
        for j in range(4):
            blk = src[:, j * PACK_ROWS:(j + 1) * PACK_ROWS]
            t = lax.dot_general(blk, eye, (((0,), (0,)), ((), ())),
                                preferred_element_type=jnp.float32)
            dst[:, j * D:(j + 1) * D] = t


def _untile(uT, iT):
    spec_in = pl.BlockSpec((D, UNTILE_W), lambda i: (0, i))
    spec_out = pl.BlockSpec((PACK_ROWS, 4 * D), lambda i: (i, 0))
    return pl.pallas_call(
        _untile_body,
        grid=(UNTILE_STEPS,),
        in_specs=[spec_in, spec_in],
        out_specs=[spec_out, spec_out],
        out_shape=[jax.ShapeDtypeStruct((NPACK, 4 * D), jnp.float32)] * 2,
    )(uT, iT)


def kernel(user_indices, item_indices, user_table, item_table):
    mesh = plsc.VectorSubcoreMesh(core_axis_name="c", subcore_axis_name="s")
    cp = pltpu.CompilerParams()
    for fld, val in (("needs_layout_passes", False),
                     ("use_tc_tiling_on_sc", True)):
        if fld in pltpu.CompilerParams.__dataclass_fields__:
            cp = dataclasses.replace(cp, **{fld: val})
    sc_call = functools.partial(
        pl.kernel,
        out_type=jax.ShapeDtypeStruct((BATCH,), jnp.float32),
        mesh=mesh,
        compiler_params=cp,
        scratch_types=[
            pltpu.VMEM((BPW,), jnp.int32),
            pltpu.VMEM((BPW,), jnp.int32),
            pltpu.VMEM((BPW,), jnp.int32),
            pltpu.VMEM((BPW,), jnp.int32),
            pltpu.VMEM((CHUNK, 4 * D), jnp.float32),
            pltpu.VMEM((CHUNK, 4 * D), jnp.float32),
            pltpu.VMEM((CHUNK, 4 * D), jnp.float32),
            pltpu.VMEM((CHUNK, 4 * D), jnp.float32),
            pltpu.VMEM((BPW,), jnp.float32),
            pltpu.SemaphoreType.DMA,
            pltpu.SemaphoreType.DMA,
        ],
    )(_sc_dot_kernel)
    ui = user_indices.astype(jnp.int32)
    ii = item_indices.astype(jnp.int32)
    u128, i128 = _untile(user_table.T, item_table.T)
    uR = ((ui >> 13) << 11) + (ui & 2047)
    iR = ((ii >> 13) << 11) + (ii & 2047)
    um = (ui & 8191) >> 11
    im = (ii & 8191) >> 11
    return sc_call(uR, um, iR, im, u128, i128)
